# baseline (speedup 1.0000x reference)
import functools

import jax
import jax.numpy as jnp
from jax import lax
from jax.experimental import pallas as pl
from jax.experimental.pallas import tpu as pltpu
from jax.experimental.pallas import tpu_sc as plsc

B = 16384
D = 32
NC = 2
NS = 16
NW = NC * NS
BPW = B // NW


def _bpr_kernel(uid_hbm, pid_hbm, nid_hbm, utab_hbm, itab_hbm,
                pos_out, neg_out,
                idx_u, idx_p, idx_n, ucols, pcols, ncols,
                pos_v, neg_v, sem_u, sem_p, sem_n):
    wid = lax.axis_index("s") * NC + lax.axis_index("c")
    base = wid * BPW

    pltpu.sync_copy(uid_hbm.at[pl.ds(base, BPW)], idx_u)
    pltpu.sync_copy(pid_hbm.at[pl.ds(base, BPW)], idx_p)
    pltpu.sync_copy(nid_hbm.at[pl.ds(base, BPW)], idx_n)

    pltpu.async_copy(utab_hbm.at[:, pl.ds(base, BPW)], ucols, sem_u).wait()
    pltpu.async_copy(itab_hbm.at[:, pl.ds(base, BPW)], pcols, sem_p).wait()
    pltpu.async_copy(itab_hbm.at[:, pl.ds(base + B, BPW)], ncols, sem_n).wait()

    def body(g, carry):
        col = g * 16
        zp = jnp.zeros((16,), jnp.float32)
        zn = jnp.zeros((16,), jnp.float32)
        for d in range(D):
            u = ucols[d, pl.ds(col, 16)]
            p = pcols[d, pl.ds(col, 16)]
            n = ncols[d, pl.ds(col, 16)]
            zp = zp + u * p
            zn = zn + u * n
        pos_v[pl.ds(col, 16)] = zp
        neg_v[pl.ds(col, 16)] = zn
        return carry

    lax.fori_loop(0, BPW // 16, body, 0)

    pltpu.sync_copy(pos_v, pos_out.at[pl.ds(base, BPW)])
    pltpu.sync_copy(neg_v, neg_out.at[pl.ds(base, BPW)])


def kernel(user_ids, pos_item_ids, neg_item_ids, user_table, item_table):
    mesh = plsc.VectorSubcoreMesh(core_axis_name="c", subcore_axis_name="s")
    run = functools.partial(
        pl.kernel,
        out_type=(jax.ShapeDtypeStruct((B,), jnp.float32),
                  jax.ShapeDtypeStruct((B,), jnp.float32)),
        mesh=mesh,
        compiler_params=pltpu.CompilerParams(
            needs_layout_passes=False, use_tc_tiling_on_sc=False),
        scratch_types=[
            pltpu.VMEM((BPW,), jnp.int32),
            pltpu.VMEM((BPW,), jnp.int32),
            pltpu.VMEM((BPW,), jnp.int32),
            pltpu.VMEM((D, BPW), jnp.float32),
            pltpu.VMEM((D, BPW), jnp.float32),
            pltpu.VMEM((D, BPW), jnp.float32),
            pltpu.VMEM((BPW,), jnp.float32),
            pltpu.VMEM((BPW,), jnp.float32),
            pltpu.SemaphoreType.DMA,
            pltpu.SemaphoreType.DMA,
            pltpu.SemaphoreType.DMA,
        ],
    )(_bpr_kernel)
    return run(user_ids, pos_item_ids, neg_item_ids,
               user_table.T, item_table.T)


# speedup vs baseline: 4.8572x; 4.8572x over previous
import functools

import jax
import jax.numpy as jnp
from jax import lax
from jax.experimental import pallas as pl
from jax.experimental.pallas import tpu as pltpu
from jax.experimental.pallas import tpu_sc as plsc

B = 16384
D = 32
NC = 2
NS = 16
NW = NC * NS
BPW = B // NW


def _bpr_kernel(uid_hbm, pid_hbm, nid_hbm, utab_hbm, itab_hbm,
                pos_out, neg_out,
                idx_u, idx_p, idx_n, urows, prows, nrows,
                pos_v, neg_v, sem_u, sem_p, sem_n):
    wid = lax.axis_index("s") * NC + lax.axis_index("c")
    base = wid * BPW

    pltpu.sync_copy(uid_hbm.at[pl.ds(base, BPW)], idx_u)
    pltpu.sync_copy(pid_hbm.at[pl.ds(base, BPW)], idx_p)
    pltpu.sync_copy(nid_hbm.at[pl.ds(base, BPW)], idx_n)

    cu = pltpu.async_copy(utab_hbm.at[idx_u], urows, sem_u)
    cp = pltpu.async_copy(itab_hbm.at[idx_p], prows, sem_p)
    cn = pltpu.async_copy(itab_hbm.at[idx_n], nrows, sem_n)
    cu.wait()
    cp.wait()
    cn.wait()

    lane = lax.iota(jnp.int32, 16)

    def body(g, carry):
        base_i = g * 16
        zp = jnp.zeros((16,), jnp.float32)
        zn = jnp.zeros((16,), jnp.float32)
        for j in range(16):
            i = base_i + j
            u0, u1 = plsc.unpack(urows[i, pl.ds(0, 32)],
                                 format=plsc.PackFormat.INTERLEAVED)
            p0, p1 = plsc.unpack(prows[i, pl.ds(0, 32)],
                                 format=plsc.PackFormat.INTERLEAVED)
            n0, n1 = plsc.unpack(nrows[i, pl.ds(0, 32)],
                                 format=plsc.PackFormat.INTERLEAVED)
            ps = jnp.sum(u0 * p0 + u1 * p1)
            ns = jnp.sum(u0 * n0 + u1 * n1)
            zp = jnp.where(lane == j, ps, zp)
            zn = jnp.where(lane == j, ns, zn)
        pos_v[pl.ds(base_i, 16)] = zp
        neg_v[pl.ds(base_i, 16)] = zn
        return carry

    lax.fori_loop(0, BPW // 16, body, 0)

    pltpu.sync_copy(pos_v, pos_out.at[pl.ds(base, BPW)])
    pltpu.sync_copy(neg_v, neg_out.at[pl.ds(base, BPW)])


def kernel(user_ids, pos_item_ids, neg_item_ids, user_table, item_table):
    mesh = plsc.VectorSubcoreMesh(core_axis_name="c", subcore_axis_name="s")
    run = functools.partial(
        pl.kernel,
        out_type=(jax.ShapeDtypeStruct((B,), jnp.float32),
                  jax.ShapeDtypeStruct((B,), jnp.float32)),
        mesh=mesh,
        compiler_params=pltpu.CompilerParams(
            needs_layout_passes=False, use_tc_tiling_on_sc=False),
        scratch_types=[
            pltpu.VMEM((BPW,), jnp.int32),
            pltpu.VMEM((BPW,), jnp.int32),
            pltpu.VMEM((BPW,), jnp.int32),
            pltpu.VMEM((BPW, D), jnp.bfloat16),
            pltpu.VMEM((BPW, D), jnp.bfloat16),
            pltpu.VMEM((BPW, D), jnp.bfloat16),
            pltpu.VMEM((BPW,), jnp.float32),
            pltpu.VMEM((BPW,), jnp.float32),
            pltpu.SemaphoreType.DMA,
            pltpu.SemaphoreType.DMA,
            pltpu.SemaphoreType.DMA,
        ],
    )(_bpr_kernel)
    return run(user_ids, pos_item_ids, neg_item_ids,
               user_table.astype(jnp.bfloat16), item_table.astype(jnp.bfloat16))


# speedup vs baseline: 5.6786x; 1.1691x over previous
import functools

import jax
import jax.numpy as jnp
from jax import lax
from jax.experimental import pallas as pl
from jax.experimental.pallas import tpu as pltpu
from jax.experimental.pallas import tpu_sc as plsc

B = 16384
D = 32
NC = 2
NS = 16
NW = NC * NS
BPW = B // NW


def _bpr_kernel(uid_hbm, pid_hbm, nid_hbm, utab_hbm, itab_hbm,
                pos_out, neg_out,
                idx_u, idx_p, idx_n, urows, prows, nrows,
                pos_v, neg_v, sem_u, sem_p, sem_n):
    wid = lax.axis_index("s") * NC + lax.axis_index("c")
    base = wid * BPW

    pltpu.sync_copy(uid_hbm.at[pl.ds(base, BPW)], idx_u)
    pltpu.sync_copy(pid_hbm.at[pl.ds(base, BPW)], idx_p)
    pltpu.sync_copy(nid_hbm.at[pl.ds(base, BPW)], idx_n)

    cu = pltpu.async_copy(utab_hbm.at[idx_u], urows, sem_u)
    cp = pltpu.async_copy(itab_hbm.at[idx_p], prows, sem_p)
    cn = pltpu.async_copy(itab_hbm.at[idx_n], nrows, sem_n)
    cu.wait()
    cp.wait()
    cn.wait()

    lane = lax.iota(jnp.int32, 16)

    def body(g, carry):
        base_i = g * 16
        zp = jnp.zeros((16,), jnp.float32)
        zn = jnp.zeros((16,), jnp.float32)
        for j in range(16):
            i = base_i + j
            u0 = urows[i, pl.ds(0, 16)]
            u1 = urows[i, pl.ds(16, 16)]
            p0 = prows[i, pl.ds(0, 16)]
            p1 = prows[i, pl.ds(16, 16)]
            n0 = nrows[i, pl.ds(0, 16)]
            n1 = nrows[i, pl.ds(16, 16)]
            ps = jnp.sum(u0 * p0 + u1 * p1)
            ns = jnp.sum(u0 * n0 + u1 * n1)
            zp = jnp.where(lane == j, ps, zp)
            zn = jnp.where(lane == j, ns, zn)
        pos_v[pl.ds(base_i, 16)] = zp
        neg_v[pl.ds(base_i, 16)] = zn
        return carry

    lax.fori_loop(0, BPW // 16, body, 0)

    pltpu.sync_copy(pos_v, pos_out.at[pl.ds(base, BPW)])
    pltpu.sync_copy(neg_v, neg_out.at[pl.ds(base, BPW)])


def kernel(user_ids, pos_item_ids, neg_item_ids, user_table, item_table):
    mesh = plsc.VectorSubcoreMesh(core_axis_name="c", subcore_axis_name="s")
    run = functools.partial(
        pl.kernel,
        out_type=(jax.ShapeDtypeStruct((B,), jnp.float32),
                  jax.ShapeDtypeStruct((B,), jnp.float32)),
        mesh=mesh,
        compiler_params=pltpu.CompilerParams(
            needs_layout_passes=False, use_tc_tiling_on_sc=False),
        scratch_types=[
            pltpu.VMEM((BPW,), jnp.int32),
            pltpu.VMEM((BPW,), jnp.int32),
            pltpu.VMEM((BPW,), jnp.int32),
            pltpu.VMEM((BPW, D), jnp.float32),
            pltpu.VMEM((BPW, D), jnp.float32),
            pltpu.VMEM((BPW, D), jnp.float32),
            pltpu.VMEM((BPW,), jnp.float32),
            pltpu.VMEM((BPW,), jnp.float32),
            pltpu.SemaphoreType.DMA,
            pltpu.SemaphoreType.DMA,
            pltpu.SemaphoreType.DMA,
        ],
    )(_bpr_kernel)
    return run(user_ids, pos_item_ids, neg_item_ids, user_table, item_table)
